# baseline (speedup 1.0000x reference)
import functools

import jax
import jax.numpy as jnp
from jax import lax
from jax.experimental import pallas as pl
from jax.experimental.pallas import tpu as pltpu
from jax.experimental.pallas import tpu_sc as plsc

W = 1920
H = 1080
NB = 135
BPX = W * 8
CAP = 768
ROWS_PER_BAND = (CAP // 32) * 1024
DUMP = NB * ROWS_PER_BAND
TOT_ROWS = DUMP + 1024
PLANE = BPX + 2048
NP = 1003520
PER_TILE = NP // 32
K3_CHUNK = 2240
L = 16


def _project_kernel(mx, my, mz, cr, cg, cb, prm, bino, lido, zo, wao, wbo, wco,
                    wdo, ro, go, bo):
    R00, R01, R02 = prm[0], prm[1], prm[2]
    R10, R11, R12 = prm[3], prm[4], prm[5]
    R20, R21, R22 = prm[6], prm[7], prm[8]
    t0, t1, t2 = prm[9], prm[10], prm[11]
    fx, fy, cx, cy = prm[12], prm[13], prm[14], prm[15]
    def bf(v):
        return v.astype(jnp.bfloat16).astype(jnp.float32)

    x_, y_, z_ = bf(mx[...]), bf(my[...]), bf(mz[...])
    R00, R01, R02, R10, R11, R12, R20, R21, R22 = [
        bf(v) for v in (R00, R01, R02, R10, R11, R12, R20, R21, R22)]
    mcx = x_ * R00 + y_ * R01 + z_ * R02 + t0
    mcy = x_ * R10 + y_ * R11 + z_ * R12 + t1
    mcz = x_ * R20 + y_ * R21 + z_ * R22 + t2
    valid = mcz > 0.1
    zs = jnp.where(valid, mcz, 1.0)
    x = mcx * fx / zs + cx
    y = mcy * fy / zs + cy
    inb = valid & (x >= 0) & (x < W - 1) & (y >= 0) & (y < H - 1)
    xc = jnp.clip(x, 0.0, float(W - 1))
    yc = jnp.clip(y, 0.0, float(H - 1))
    x0f = jnp.floor(xc)
    y0f = jnp.floor(yc)
    ix = x0f.astype(jnp.int32)
    iy = y0f.astype(jnp.int32)
    dx = xc - x0f
    dy = yc - y0f
    bino[...] = jnp.where(inb, iy >> 3, -1)
    lido[...] = (iy & 7) * W + ix
    zo[...] = zs
    wao[...] = (1 - dx) * (1 - dy)
    wbo[...] = dx * (1 - dy)
    wco[...] = (1 - dx) * dy
    wdo[...] = dx * dy
    ro[...] = jax.nn.sigmoid(cr[...])
    go[...] = jax.nn.sigmoid(cg[...])
    bo[...] = jax.nn.sigmoid(cb[...])


def _run_project(mx, my, mz, cr, cg, cb, prm):
    grid = 10
    blk = NP // 128 // grid
    spec = pl.BlockSpec((blk, 128), lambda i: (i, 0))
    out10 = pl.pallas_call(
        _project_kernel,
        grid=(grid,),
        in_specs=[spec] * 6 + [pl.BlockSpec(memory_space=pltpu.SMEM)],
        out_specs=[spec] * 10,
        out_shape=(
            [jax.ShapeDtypeStruct((NP // 128, 128), jnp.int32)] * 2
            + [jax.ShapeDtypeStruct((NP // 128, 128), jnp.float32)] * 8
        ),
    )(mx, my, mz, cr, cg, cb, prm)
    return out10


_sc_mesh = plsc.VectorSubcoreMesh(core_axis_name="c", subcore_axis_name="s")


@functools.partial(
    pl.kernel,
    out_type=(
        jax.ShapeDtypeStruct((TOT_ROWS, L), jnp.float32),
        jax.ShapeDtypeStruct((32, 136), jnp.int32),
    ),
    mesh=_sc_mesh,
    compiler_params=pltpu.CompilerParams(needs_layout_passes=False, use_tc_tiling_on_sc=False),
    scratch_types=[
        pltpu.VMEM((144,), jnp.int32),
        pltpu.VMEM((L,), jnp.int32),
        [pltpu.VMEM((K3_CHUNK,), jnp.int32)] * 2
        + [pltpu.VMEM((K3_CHUNK,), jnp.float32)] * 8,
        pltpu.VMEM((K3_CHUNK, L), jnp.float32),
        pltpu.VMEM((K3_CHUNK,), jnp.int32),
        pltpu.SemaphoreType.DMA,
        pltpu.SemaphoreType.DMA,
    ],
)
def _bin_kernel(binh, lidh, zh, wah, wbh, wch, wdh, rh, gh, bh,
                rows_hbm, counts_hbm, counters, tmp16, stage_in, rows_v,
                dest_v, sem_in, sem_out):
    t = lax.axis_index("s") * 2 + lax.axis_index("c")
    base_pt = t * PER_TILE
    iota = lax.iota(jnp.int32, L)

    def zero_counters(i, _):
        counters[pl.ds(i * L, L)] = jnp.zeros((L,), jnp.int32)
        return 0

    lax.fori_loop(0, 144 // L, zero_counters, 0)

    inputs = (binh, lidh, zh, wah, wbh, wch, wdh, rh, gh, bh)

    for ci in range(PER_TILE // K3_CHUNK):
        start = base_pt + ci * K3_CHUNK
        cps = [
            pltpu.async_copy(inp.at[pl.ds(start, K3_CHUNK)], stage_in[f], sem_in)
            for f, inp in enumerate(inputs)
        ]
        for cp in cps:
            cp.wait()

        def vbody(vi, _):
            b16 = stage_in[0][pl.ds(vi * L, L)]
            sbin, perm = plsc.sort_key_val(b16, iota)
            prev = jnp.take(sbin, jnp.maximum(iota - 1, 0), mode="wrap")
            isstart = (sbin != prev) | (iota == 0)
            spos = plsc.cummax(jnp.where(isstart, iota, 0))
            occ_s = iota - spos
            nxt = jnp.take(sbin, jnp.minimum(iota + 1, L - 1), mode="wrap")
            islast = (sbin != nxt) | (iota == L - 1)
            sbc = jnp.where(sbin < 0, 135, sbin)
            base_s = plsc.load_gather(counters, [sbc])
            plsc.store_scatter(counters, [sbc], base_s + occ_s + 1, mask=islast)
            r_s = base_s + occ_s
            plsc.store_scatter(tmp16, [perm], r_s)
            r = tmp16[...]
            slot = (b16 * ROWS_PER_BAND + (r >> 5) * 1024 + t * 32 + (r & 31))
            ok = (b16 >= 0) & (r < CAP)
            dump = DUMP + ((t * 32 + (vi * L + iota)) & 1023)
            dest_v[pl.ds(vi * L, L)] = jnp.where(ok, slot, dump)
            rowpos = vi * L + iota
            lid = stage_in[1][pl.ds(vi * L, L)]
            plsc.store_scatter(rows_v, [rowpos, jnp.zeros((L,), jnp.int32)],
                               lid.astype(jnp.float32))
            for f in range(2, 10):
                val = stage_in[f][pl.ds(vi * L, L)]
                plsc.store_scatter(rows_v, [rowpos, jnp.full((L,), f - 1, jnp.int32)], val)
            return 0

        lax.fori_loop(0, K3_CHUNK // L, vbody, 0)
        pltpu.async_copy(rows_v, rows_hbm.at[dest_v], sem_out).wait()

    pltpu.sync_copy(counters.at[pl.ds(0, 136)], counts_hbm.at[t])


@functools.partial(
    pl.kernel,
    out_type=(
        jax.ShapeDtypeStruct((NB, 5, BPX), jnp.float32),
        jax.ShapeDtypeStruct((NB, 5, 2048), jnp.float32),
    ),
    mesh=_sc_mesh,
    compiler_params=pltpu.CompilerParams(needs_layout_passes=False, use_tc_tiling_on_sc=False),
    scratch_types=[
        pltpu.VMEM((5 * PLANE,), jnp.float32),
        pltpu.VMEM((BPX,), jnp.float32),
        pltpu.VMEM((512, L), jnp.float32),
        pltpu.VMEM((32, 136), jnp.int32),
        pltpu.SemaphoreType.DMA,
    ],
)
def _raster_kernel(rows_hbm, counts_hbm, main_hbm, halo_hbm,
                   splat, zband, rows_v, counts_v, sem):
    t = lax.axis_index("s") * 2 + lax.axis_index("c")
    iota = lax.iota(jnp.int32, L)
    pltpu.sync_copy(counts_hbm, counts_v)

    def band_body(k, _):
        b = t + 32 * k

        def do_band():
            bandbase = b * ROWS_PER_BAND

            def zzero(i, _):
                zband[pl.ds(i * L, L)] = jnp.full((L,), 100.0, jnp.float32)
                return 0

            lax.fori_loop(0, BPX // L, zzero, 0)

            def szero(i, _):
                splat[pl.ds(i * L, L)] = jnp.zeros((L,), jnp.float32)
                return 0

            lax.fori_loop(0, 5 * PLANE // L, szero, 0)

            c0 = plsc.load_gather(counts_v, [iota, jnp.full((L,), b, jnp.int32)])
            c1 = plsc.load_gather(counts_v, [iota + 16, jnp.full((L,), b, jnp.int32)])
            cmax = jnp.max(jnp.maximum(c0, c1))
            cmax = jnp.minimum(cmax, CAP)
            n512 = ((cmax + 31) // 32) * 2

            def zchunk(c5, _):
                pltpu.sync_copy(rows_hbm.at[pl.ds(bandbase + c5 * 512, 512)], rows_v)

                def zvec(vi, _):
                    jvec = c5 * 512 + vi * L + iota
                    t16 = (jvec >> 5) & 31
                    r16 = (jvec >> 10) * 32 + (jvec & 31)
                    cnt = plsc.load_gather(counts_v, [t16, jnp.full((L,), b, jnp.int32)])
                    ok = r16 < jnp.minimum(cnt, CAP)
                    loc = vi * L + iota
                    lid = plsc.load_gather(rows_v, [loc, jnp.zeros((L,), jnp.int32)])
                    z = plsc.load_gather(rows_v, [loc, jnp.ones((L,), jnp.int32)])
                    lidc = jnp.clip(lid.astype(jnp.int32), 0, BPX - 1)
                    zm = jnp.where(ok, z, 1e9)
                    sl, sz = plsc.sort_key_val(lidc, zm)
                    for kk in (1, 2, 4, 8):
                        pk = jnp.maximum(iota - kk, 0)
                        same = jnp.take(sl, pk, mode="wrap") == sl
                        cand = jnp.take(sz, pk, mode="wrap")
                        sz = jnp.where(same & (iota >= kk), jnp.minimum(sz, cand), sz)
                    nxt = jnp.take(sl, jnp.minimum(iota + 1, L - 1), mode="wrap")
                    islast = (sl != nxt) | (iota == L - 1)
                    cur = plsc.load_gather(zband, [sl])
                    plsc.store_scatter(zband, [sl], jnp.minimum(sz, cur), mask=islast)
                    return 0

                lax.fori_loop(0, 32, zvec, 0)
                return 0

            lax.fori_loop(0, n512, zchunk, 0)

            def schunk(c5, _):
                pltpu.sync_copy(rows_hbm.at[pl.ds(bandbase + c5 * 512, 512)], rows_v)

                def svec(vi, _):
                    jvec = c5 * 512 + vi * L + iota
                    t16 = (jvec >> 5) & 31
                    r16 = (jvec >> 10) * 32 + (jvec & 31)
                    cnt = plsc.load_gather(counts_v, [t16, jnp.full((L,), b, jnp.int32)])
                    ok = r16 < jnp.minimum(cnt, CAP)
                    loc = vi * L + iota

                    def fld(f):
                        return plsc.load_gather(rows_v, [loc, jnp.full((L,), f, jnp.int32)])

                    lid = fld(0)
                    z = fld(1)
                    lidc = jnp.clip(lid.astype(jnp.int32), 0, BPX - 1)
                    minz = plsc.load_gather(zband, [lidc])
                    vis = ok & (z <= minz + 0.05)
                    zero = jnp.zeros((L,), jnp.float32)
                    wa = jnp.where(vis, fld(2), zero)
                    wb = jnp.where(vis, fld(3), zero)
                    wc = jnp.where(vis, fld(4), zero)
                    wd = jnp.where(vis, fld(5), zero)
                    r = jnp.where(vis, fld(6), zero)
                    g = jnp.where(vis, fld(7), zero)
                    bl = jnp.where(vis, fld(8), zero)
                    zf = jnp.where(vis, z, zero)
                    feats = (r, g, bl, zf)
                    for off, wgt in ((0, wa), (W, wb), (1, wc), (W + 1, wd)):
                        cidx = lidc + off
                        for ch in range(4):
                            plsc.addupdate_scatter(
                                splat, [cidx + ch * PLANE], wgt * feats[ch])
                        plsc.addupdate_scatter(splat, [cidx + 4 * PLANE], wgt)
                    return 0

                lax.fori_loop(0, 32, svec, 0)
                return 0

            lax.fori_loop(0, n512, schunk, 0)

            for ch in range(5):
                pltpu.sync_copy(splat.at[pl.ds(ch * PLANE, BPX)], main_hbm.at[b, ch])
                pltpu.sync_copy(splat.at[pl.ds(ch * PLANE + BPX, 2048)],
                                halo_hbm.at[b, ch])

        pl.when(b < NB)(do_band)
        return 0

    lax.fori_loop(0, 5, band_body, 0)


def _normalize_kernel(main, halo, out):
    pid = pl.program_id(0)
    m = main[...].reshape(5, BPX)
    hp = halo[...].reshape(5, 2048)
    factor = jnp.where(pid > 0, 1.0, 0.0)
    hpad = jnp.pad(hp * factor, ((0, 0), (0, BPX - 2048)))
    acc = m + hpad
    tw = acc[4] + 1e-6
    inv = 1.0 / tw
    rr = jnp.clip(acc[0] * inv, 0.0, 1.0)
    gg = jnp.clip(acc[1] * inv, 0.0, 1.0)
    bb = jnp.clip(acc[2] * inv, 0.0, 1.0)
    dd = acc[3] * inv
    out[...] = jnp.stack([rr, gg, bb, dd], axis=-1).reshape(8, W, 4)


def _run_normalize(main, halo):
    return pl.pallas_call(
        _normalize_kernel,
        grid=(NB,),
        in_specs=[
            pl.BlockSpec((1, 5, BPX), lambda i: (i, 0, 0)),
            pl.BlockSpec((1, 5, 2048), lambda i: (jnp.maximum(i - 1, 0), 0, 0)),
        ],
        out_specs=pl.BlockSpec((8, W, 4), lambda i: (i, 0, 0)),
        out_shape=jax.ShapeDtypeStruct((H, W, 4), jnp.float32),
    )(main, halo)


def kernel(means, colors, opacities, scales, quats, viewmat, K, height, width):
    n = means.shape[0]
    pad = NP - n
    mx = jnp.concatenate([means[:, 0], jnp.zeros((pad,), jnp.float32)])
    my = jnp.concatenate([means[:, 1], jnp.zeros((pad,), jnp.float32)])
    mz = jnp.concatenate([means[:, 2], jnp.full((pad,), -100.0, jnp.float32)])
    cr = jnp.concatenate([colors[:, 0], jnp.zeros((pad,), jnp.float32)])
    cg = jnp.concatenate([colors[:, 1], jnp.zeros((pad,), jnp.float32)])
    cb = jnp.concatenate([colors[:, 2], jnp.zeros((pad,), jnp.float32)])
    shp = (NP // 128, 128)
    R = viewmat[:3, :3]
    tv = viewmat[:3, 3]
    prm = jnp.concatenate([
        R.reshape(9), tv.reshape(3),
        jnp.stack([K[0, 0], K[1, 1], K[0, 2], K[1, 2]]),
    ]).astype(jnp.float32)
    outs = _run_project(mx.reshape(shp), my.reshape(shp), mz.reshape(shp),
                        cr.reshape(shp), cg.reshape(shp), cb.reshape(shp), prm)
    flat = [o.reshape(NP) for o in outs]
    rows, counts = _bin_kernel(*flat)
    main, halo = _raster_kernel(rows, counts)
    return _run_normalize(main, halo)


# speedup vs baseline: 2.1919x; 2.1919x over previous
import functools

import jax
import jax.numpy as jnp
from jax import lax
from jax.experimental import pallas as pl
from jax.experimental.pallas import tpu as pltpu
from jax.experimental.pallas import tpu_sc as plsc

W = 1920
H = 1080
NB = 135
BPX = W * 8
CAP = 768
ROWS_PER_BAND = (CAP // 32) * 1024
DUMP = NB * ROWS_PER_BAND
TOT_ROWS = DUMP + 1024
PLANE = BPX + 2048
NP = 1003520
PER_TILE = NP // 32
K3_CHUNK = 2240
L = 16


def _project_kernel(mx, my, mz, cr, cg, cb, prm, bino, lido, zo, wao, wbo, wco,
                    wdo, ro, go, bo):
    R00, R01, R02 = prm[0], prm[1], prm[2]
    R10, R11, R12 = prm[3], prm[4], prm[5]
    R20, R21, R22 = prm[6], prm[7], prm[8]
    t0, t1, t2 = prm[9], prm[10], prm[11]
    fx, fy, cx, cy = prm[12], prm[13], prm[14], prm[15]
    def bf(v):
        return v.astype(jnp.bfloat16).astype(jnp.float32)

    x_, y_, z_ = bf(mx[...]), bf(my[...]), bf(mz[...])
    R00, R01, R02, R10, R11, R12, R20, R21, R22 = [
        bf(v) for v in (R00, R01, R02, R10, R11, R12, R20, R21, R22)]
    mcx = x_ * R00 + y_ * R01 + z_ * R02 + t0
    mcy = x_ * R10 + y_ * R11 + z_ * R12 + t1
    mcz = x_ * R20 + y_ * R21 + z_ * R22 + t2
    valid = mcz > 0.1
    zs = jnp.where(valid, mcz, 1.0)
    x = mcx * fx / zs + cx
    y = mcy * fy / zs + cy
    inb = valid & (x >= 0) & (x < W - 1) & (y >= 0) & (y < H - 1)
    xc = jnp.clip(x, 0.0, float(W - 1))
    yc = jnp.clip(y, 0.0, float(H - 1))
    x0f = jnp.floor(xc)
    y0f = jnp.floor(yc)
    ix = x0f.astype(jnp.int32)
    iy = y0f.astype(jnp.int32)
    dx = xc - x0f
    dy = yc - y0f
    bino[...] = jnp.where(inb, iy >> 3, -1)
    lido[...] = (iy & 7) * W + ix
    zo[...] = zs
    wao[...] = (1 - dx) * (1 - dy)
    wbo[...] = dx * (1 - dy)
    wco[...] = (1 - dx) * dy
    wdo[...] = dx * dy
    ro[...] = jax.nn.sigmoid(cr[...])
    go[...] = jax.nn.sigmoid(cg[...])
    bo[...] = jax.nn.sigmoid(cb[...])


def _run_project(mx, my, mz, cr, cg, cb, prm):
    grid = 10
    blk = NP // 128 // grid
    spec = pl.BlockSpec((blk, 128), lambda i: (i, 0))
    out10 = pl.pallas_call(
        _project_kernel,
        grid=(grid,),
        in_specs=[spec] * 6 + [pl.BlockSpec(memory_space=pltpu.SMEM)],
        out_specs=[spec] * 10,
        out_shape=(
            [jax.ShapeDtypeStruct((NP // 128, 128), jnp.int32)] * 2
            + [jax.ShapeDtypeStruct((NP // 128, 128), jnp.float32)] * 8
        ),
    )(mx, my, mz, cr, cg, cb, prm)
    return out10


_sc_mesh = plsc.VectorSubcoreMesh(core_axis_name="c", subcore_axis_name="s")


@functools.partial(
    pl.kernel,
    out_type=(
        jax.ShapeDtypeStruct((TOT_ROWS, L), jnp.float32),
        jax.ShapeDtypeStruct((32, 136), jnp.int32),
    ),
    mesh=_sc_mesh,
    compiler_params=pltpu.CompilerParams(needs_layout_passes=False, use_tc_tiling_on_sc=False),
    scratch_types=[
        pltpu.VMEM((144,), jnp.int32),
        pltpu.VMEM((L,), jnp.int32),
        [pltpu.VMEM((K3_CHUNK,), jnp.int32)] * 2
        + [pltpu.VMEM((K3_CHUNK,), jnp.float32)] * 8,
        pltpu.VMEM((K3_CHUNK, L), jnp.float32),
        pltpu.VMEM((K3_CHUNK,), jnp.int32),
        pltpu.SemaphoreType.DMA,
        pltpu.SemaphoreType.DMA,
    ],
)
def _bin_kernel(binh, lidh, zh, wah, wbh, wch, wdh, rh, gh, bh,
                rows_hbm, counts_hbm, counters, tmp16, stage_in, rows_v,
                dest_v, sem_in, sem_out):
    t = lax.axis_index("s") * 2 + lax.axis_index("c")
    base_pt = t * PER_TILE
    iota = lax.iota(jnp.int32, L)

    def zero_counters(i, _):
        counters[pl.ds(i * L, L)] = jnp.zeros((L,), jnp.int32)
        return 0

    lax.fori_loop(0, 144 // L, zero_counters, 0)

    inputs = (binh, lidh, zh, wah, wbh, wch, wdh, rh, gh, bh)

    for ci in range(PER_TILE // K3_CHUNK):
        start = base_pt + ci * K3_CHUNK
        cps = [
            pltpu.async_copy(inp.at[pl.ds(start, K3_CHUNK)], stage_in[f], sem_in)
            for f, inp in enumerate(inputs)
        ]
        for cp in cps:
            cp.wait()

        def vbody(vi, _):
            b16 = stage_in[0][pl.ds(vi * L, L)]
            sbin, perm = plsc.sort_key_val(b16, iota)
            prev = jnp.take(sbin, jnp.maximum(iota - 1, 0), mode="wrap")
            isstart = (sbin != prev) | (iota == 0)
            spos = plsc.cummax(jnp.where(isstart, iota, 0))
            occ_s = iota - spos
            nxt = jnp.take(sbin, jnp.minimum(iota + 1, L - 1), mode="wrap")
            islast = (sbin != nxt) | (iota == L - 1)
            sbc = jnp.where(sbin < 0, 135, sbin)
            base_s = plsc.load_gather(counters, [sbc])
            plsc.store_scatter(counters, [sbc], base_s + occ_s + 1, mask=islast)
            r_s = base_s + occ_s
            plsc.store_scatter(tmp16, [perm], r_s)
            r = tmp16[...]
            slot = (b16 * ROWS_PER_BAND + (r >> 5) * 1024 + t * 32 + (r & 31))
            ok = (b16 >= 0) & (r < CAP)
            dump = DUMP + ((t * 32 + (vi * L + iota)) & 1023)
            dest_v[pl.ds(vi * L, L)] = jnp.where(ok, slot, dump)
            rowpos = vi * L + iota
            lid = stage_in[1][pl.ds(vi * L, L)]
            plsc.store_scatter(rows_v, [rowpos, jnp.zeros((L,), jnp.int32)],
                               lid.astype(jnp.float32))
            for f in range(2, 10):
                val = stage_in[f][pl.ds(vi * L, L)]
                plsc.store_scatter(rows_v, [rowpos, jnp.full((L,), f - 1, jnp.int32)], val)
            return 0

        lax.fori_loop(0, K3_CHUNK // L, vbody, 0)
        pltpu.async_copy(rows_v, rows_hbm.at[dest_v], sem_out).wait()

    pltpu.sync_copy(counters.at[pl.ds(0, 136)], counts_hbm.at[t])


@functools.partial(
    pl.kernel,
    out_type=(
        jax.ShapeDtypeStruct((NB, 5, BPX), jnp.float32),
        jax.ShapeDtypeStruct((NB, 5, 2048), jnp.float32),
    ),
    mesh=_sc_mesh,
    compiler_params=pltpu.CompilerParams(needs_layout_passes=False, use_tc_tiling_on_sc=False),
    scratch_types=[
        pltpu.VMEM((5 * PLANE,), jnp.float32),
        pltpu.VMEM((BPX,), jnp.float32),
        pltpu.VMEM((512, L), jnp.float32),
        pltpu.VMEM((512, L), jnp.float32),
        pltpu.VMEM((32, 136), jnp.int32),
        pltpu.SemaphoreType.DMA,
        pltpu.SemaphoreType.DMA,
    ],
)
def _raster_kernel(rows_hbm, counts_hbm, main_hbm, halo_hbm,
                   splat, zband, rows_va, rows_vb, counts_v, sem_a, sem_b):
    t = lax.axis_index("s") * 2 + lax.axis_index("c")
    iota = lax.iota(jnp.int32, L)
    pltpu.sync_copy(counts_hbm, counts_v)

    def band_body(k, _):
        b = t + 32 * k

        def do_band():
            bandbase = b * ROWS_PER_BAND

            def zzero(i, _):
                zband[pl.ds(i * L, L)] = jnp.full((L,), 100.0, jnp.float32)
                return 0

            lax.fori_loop(0, BPX // L, zzero, 0)

            def szero(i, _):
                splat[pl.ds(i * L, L)] = jnp.zeros((L,), jnp.float32)
                return 0

            lax.fori_loop(0, 5 * PLANE // L, szero, 0)

            c0 = plsc.load_gather(counts_v, [iota, jnp.full((L,), b, jnp.int32)])
            c1 = plsc.load_gather(counts_v, [iota + 16, jnp.full((L,), b, jnp.int32)])
            cmax = jnp.max(jnp.maximum(c0, c1))
            cmax = jnp.minimum(cmax, CAP)
            n512 = ((cmax + 31) // 32) * 2

            def pipelined(process):
                def proc(ref, c5):
                    def body(vi, _):
                        process(ref, c5, vi)
                        return 0
                    lax.fori_loop(0, 32, body, 0)

                def fetch(c5, ref, sem):
                    pltpu.async_copy(
                        rows_hbm.at[pl.ds(bandbase + c5 * 512, 512)], ref, sem)

                def waitbuf(ref, sem):
                    pltpu.make_async_copy(
                        rows_hbm.at[pl.ds(bandbase, 512)], ref, sem).wait()

                fetch(0, rows_va, sem_a)

                def pair(i, _):
                    c0 = 2 * i
                    waitbuf(rows_va, sem_a)
                    fetch(c0 + 1, rows_vb, sem_b)
                    proc(rows_va, c0)
                    waitbuf(rows_vb, sem_b)
                    fetch(c0 + 2, rows_va, sem_a)
                    pl.when(c0 + 1 < n512)(lambda: proc(rows_vb, c0 + 1))
                    return 0

                lax.fori_loop(0, (n512 + 1) // 2, pair, 0)
                waitbuf(rows_va, sem_a)

            def zvec(rows_v, c5, vi):
                if True:
                    jvec = c5 * 512 + vi * L + iota
                    t16 = (jvec >> 5) & 31
                    r16 = (jvec >> 10) * 32 + (jvec & 31)
                    cnt = plsc.load_gather(counts_v, [t16, jnp.full((L,), b, jnp.int32)])
                    ok = r16 < jnp.minimum(cnt, CAP)
                    loc = vi * L + iota
                    lid = plsc.load_gather(rows_v, [loc, jnp.zeros((L,), jnp.int32)])
                    z = plsc.load_gather(rows_v, [loc, jnp.ones((L,), jnp.int32)])
                    lidc = jnp.clip(lid.astype(jnp.int32), 0, BPX - 1)
                    zm = jnp.where(ok, z, 1e9)
                    sl, sz = plsc.sort_key_val(lidc, zm)
                    for kk in (1, 2, 4, 8):
                        pk = jnp.maximum(iota - kk, 0)
                        same = jnp.take(sl, pk, mode="wrap") == sl
                        cand = jnp.take(sz, pk, mode="wrap")
                        sz = jnp.where(same & (iota >= kk), jnp.minimum(sz, cand), sz)
                    nxt = jnp.take(sl, jnp.minimum(iota + 1, L - 1), mode="wrap")
                    islast = (sl != nxt) | (iota == L - 1)
                    cur = plsc.load_gather(zband, [sl])
                    plsc.store_scatter(zband, [sl], jnp.minimum(sz, cur), mask=islast)

            pipelined(zvec)

            def svec(rows_v, c5, vi):
                if True:
                    jvec = c5 * 512 + vi * L + iota
                    t16 = (jvec >> 5) & 31
                    r16 = (jvec >> 10) * 32 + (jvec & 31)
                    cnt = plsc.load_gather(counts_v, [t16, jnp.full((L,), b, jnp.int32)])
                    ok = r16 < jnp.minimum(cnt, CAP)
                    loc = vi * L + iota

                    def fld(f):
                        return plsc.load_gather(rows_v, [loc, jnp.full((L,), f, jnp.int32)])

                    lid = fld(0)
                    z = fld(1)
                    lidc = jnp.clip(lid.astype(jnp.int32), 0, BPX - 1)
                    minz = plsc.load_gather(zband, [lidc])
                    vis = ok & (z <= minz + 0.05)
                    zero = jnp.zeros((L,), jnp.float32)
                    wa = jnp.where(vis, fld(2), zero)
                    wb = jnp.where(vis, fld(3), zero)
                    wc = jnp.where(vis, fld(4), zero)
                    wd = jnp.where(vis, fld(5), zero)
                    r = jnp.where(vis, fld(6), zero)
                    g = jnp.where(vis, fld(7), zero)
                    bl = jnp.where(vis, fld(8), zero)
                    zf = jnp.where(vis, z, zero)
                    feats = (r, g, bl, zf)
                    for off, wgt in ((0, wa), (W, wb), (1, wc), (W + 1, wd)):
                        cidx = lidc + off
                        for ch in range(4):
                            plsc.addupdate_scatter(
                                splat, [cidx + ch * PLANE], wgt * feats[ch])
                        plsc.addupdate_scatter(splat, [cidx + 4 * PLANE], wgt)

            pipelined(svec)

            for ch in range(5):
                pltpu.sync_copy(splat.at[pl.ds(ch * PLANE, BPX)], main_hbm.at[b, ch])
                pltpu.sync_copy(splat.at[pl.ds(ch * PLANE + BPX, 2048)],
                                halo_hbm.at[b, ch])

        pl.when(b < NB)(do_band)
        return 0

    lax.fori_loop(0, 5, band_body, 0)


def _normalize_kernel(main, halo, out):
    pid = pl.program_id(0)
    m = main[...].reshape(5, BPX)
    hp = halo[...].reshape(5, 2048)
    factor = jnp.where(pid > 0, 1.0, 0.0)
    hpad = jnp.pad(hp * factor, ((0, 0), (0, BPX - 2048)))
    acc = m + hpad
    tw = acc[4] + 1e-6
    inv = 1.0 / tw
    rr = jnp.clip(acc[0] * inv, 0.0, 1.0)
    gg = jnp.clip(acc[1] * inv, 0.0, 1.0)
    bb = jnp.clip(acc[2] * inv, 0.0, 1.0)
    dd = acc[3] * inv
    out[...] = jnp.stack([rr, gg, bb, dd], axis=0).reshape(1, 4, BPX)


def _run_normalize(main, halo):
    return pl.pallas_call(
        _normalize_kernel,
        grid=(NB,),
        in_specs=[
            pl.BlockSpec((1, 5, BPX), lambda i: (i, 0, 0)),
            pl.BlockSpec((1, 5, 2048), lambda i: (jnp.maximum(i - 1, 0), 0, 0)),
        ],
        out_specs=pl.BlockSpec((1, 4, BPX), lambda i: (i, 0, 0)),
        out_shape=jax.ShapeDtypeStruct((NB, 4, BPX), jnp.float32),
    )(main, halo)


def kernel(means, colors, opacities, scales, quats, viewmat, K, height, width):
    n = means.shape[0]
    pad = NP - n
    mx = jnp.concatenate([means[:, 0], jnp.zeros((pad,), jnp.float32)])
    my = jnp.concatenate([means[:, 1], jnp.zeros((pad,), jnp.float32)])
    mz = jnp.concatenate([means[:, 2], jnp.full((pad,), -100.0, jnp.float32)])
    cr = jnp.concatenate([colors[:, 0], jnp.zeros((pad,), jnp.float32)])
    cg = jnp.concatenate([colors[:, 1], jnp.zeros((pad,), jnp.float32)])
    cb = jnp.concatenate([colors[:, 2], jnp.zeros((pad,), jnp.float32)])
    shp = (NP // 128, 128)
    R = viewmat[:3, :3]
    tv = viewmat[:3, 3]
    prm = jnp.concatenate([
        R.reshape(9), tv.reshape(3),
        jnp.stack([K[0, 0], K[1, 1], K[0, 2], K[1, 2]]),
    ]).astype(jnp.float32)
    outs = _run_project(mx.reshape(shp), my.reshape(shp), mz.reshape(shp),
                        cr.reshape(shp), cg.reshape(shp), cb.reshape(shp), prm)
    flat = [o.reshape(NP) for o in outs]
    rows, counts = _bin_kernel(*flat)
    main, halo = _raster_kernel(rows, counts)
    outp = _run_normalize(main, halo)
    return jnp.transpose(outp.reshape(NB, 4, 8, W), (0, 2, 3, 1)).reshape(H, W, 4)


# speedup vs baseline: 2.2907x; 1.0451x over previous
import functools

import jax
import jax.numpy as jnp
from jax import lax
from jax.experimental import pallas as pl
from jax.experimental.pallas import tpu as pltpu
from jax.experimental.pallas import tpu_sc as plsc

W = 1920
H = 1080
NB = 135
BPX = W * 8
CAP = 768
ROWS_PER_BAND = (CAP // 32) * 1024
DUMP = NB * ROWS_PER_BAND
TOT_ROWS = DUMP + 1024
PLANE = BPX + 2048
NP = 1003520
PER_TILE = NP // 32
K3_CHUNK = 1120
L = 16


def _project_kernel(mx, my, mz, cr, cg, cb, prm, bino, lido, zo, wao, wbo, wco,
                    wdo, ro, go, bo):
    R00, R01, R02 = prm[0], prm[1], prm[2]
    R10, R11, R12 = prm[3], prm[4], prm[5]
    R20, R21, R22 = prm[6], prm[7], prm[8]
    t0, t1, t2 = prm[9], prm[10], prm[11]
    fx, fy, cx, cy = prm[12], prm[13], prm[14], prm[15]
    def bf(v):
        return v.astype(jnp.bfloat16).astype(jnp.float32)

    x_, y_, z_ = bf(mx[...]), bf(my[...]), bf(mz[...])
    R00, R01, R02, R10, R11, R12, R20, R21, R22 = [
        bf(v) for v in (R00, R01, R02, R10, R11, R12, R20, R21, R22)]
    mcx = x_ * R00 + y_ * R01 + z_ * R02 + t0
    mcy = x_ * R10 + y_ * R11 + z_ * R12 + t1
    mcz = x_ * R20 + y_ * R21 + z_ * R22 + t2
    valid = mcz > 0.1
    zs = jnp.where(valid, mcz, 1.0)
    x = mcx * fx / zs + cx
    y = mcy * fy / zs + cy
    inb = valid & (x >= 0) & (x < W - 1) & (y >= 0) & (y < H - 1)
    xc = jnp.clip(x, 0.0, float(W - 1))
    yc = jnp.clip(y, 0.0, float(H - 1))
    x0f = jnp.floor(xc)
    y0f = jnp.floor(yc)
    ix = x0f.astype(jnp.int32)
    iy = y0f.astype(jnp.int32)
    dx = xc - x0f
    dy = yc - y0f
    bino[...] = jnp.where(inb, iy >> 3, -1)
    lido[...] = (iy & 7) * W + ix
    zo[...] = zs
    wao[...] = (1 - dx) * (1 - dy)
    wbo[...] = dx * (1 - dy)
    wco[...] = (1 - dx) * dy
    wdo[...] = dx * dy
    ro[...] = jax.nn.sigmoid(cr[...])
    go[...] = jax.nn.sigmoid(cg[...])
    bo[...] = jax.nn.sigmoid(cb[...])


def _run_project(mx, my, mz, cr, cg, cb, prm):
    grid = 10
    blk = NP // 128 // grid
    spec = pl.BlockSpec((blk, 128), lambda i: (i, 0))
    out10 = pl.pallas_call(
        _project_kernel,
        grid=(grid,),
        in_specs=[spec] * 6 + [pl.BlockSpec(memory_space=pltpu.SMEM)],
        out_specs=[spec] * 10,
        out_shape=(
            [jax.ShapeDtypeStruct((NP // 128, 128), jnp.int32)] * 2
            + [jax.ShapeDtypeStruct((NP // 128, 128), jnp.float32)] * 8
        ),
    )(mx, my, mz, cr, cg, cb, prm)
    return out10


_sc_mesh = plsc.VectorSubcoreMesh(core_axis_name="c", subcore_axis_name="s")


@functools.partial(
    pl.kernel,
    out_type=(
        jax.ShapeDtypeStruct((TOT_ROWS, L), jnp.float32),
        jax.ShapeDtypeStruct((32, 136), jnp.int32),
    ),
    mesh=_sc_mesh,
    compiler_params=pltpu.CompilerParams(needs_layout_passes=False, use_tc_tiling_on_sc=False),
    scratch_types=[
        pltpu.VMEM((144,), jnp.int32),
        pltpu.VMEM((L,), jnp.int32),
        [pltpu.VMEM((K3_CHUNK,), jnp.int32)] * 2
        + [pltpu.VMEM((K3_CHUNK,), jnp.float32)] * 8,
        [pltpu.VMEM((K3_CHUNK,), jnp.int32)] * 2
        + [pltpu.VMEM((K3_CHUNK,), jnp.float32)] * 8,
        pltpu.VMEM((K3_CHUNK, L), jnp.float32),
        pltpu.VMEM((K3_CHUNK, L), jnp.float32),
        pltpu.VMEM((K3_CHUNK,), jnp.int32),
        pltpu.VMEM((K3_CHUNK,), jnp.int32),
        pltpu.SemaphoreType.DMA,
        pltpu.SemaphoreType.DMA,
        pltpu.SemaphoreType.DMA,
    ],
)
def _bin_kernel(binh, lidh, zh, wah, wbh, wch, wdh, rh, gh, bh,
                rows_hbm, counts_hbm, counters, tmp16, stage_a, stage_b,
                rows_va, rows_vb, dest_va, dest_vb, sem_in, sem_oa, sem_ob):
    t = lax.axis_index("s") * 2 + lax.axis_index("c")
    base_pt = t * PER_TILE
    iota = lax.iota(jnp.int32, L)

    def zero_counters(i, _):
        counters[pl.ds(i * L, L)] = jnp.zeros((L,), jnp.int32)
        return 0

    lax.fori_loop(0, 144 // L, zero_counters, 0)

    inputs = (binh, lidh, zh, wah, wbh, wch, wdh, rh, gh, bh)
    sets = ((stage_a, rows_va, dest_va, sem_oa), (stage_b, rows_vb, dest_vb, sem_ob))
    nchunks = PER_TILE // K3_CHUNK

    def fire_in(ci, stage):
        start = base_pt + ci * K3_CHUNK
        return [
            pltpu.async_copy(inp.at[pl.ds(start, K3_CHUNK)], stage[f], sem_in)
            for f, inp in enumerate(inputs)
        ]

    cps = fire_in(0, stage_a)
    cp_out = [None, None]
    for ci in range(nchunks):
        stage_in, rows_v, dest_v, sem_out = sets[ci % 2]
        for cp in cps:
            cp.wait()
        if ci + 1 < nchunks:
            cps = fire_in(ci + 1, sets[(ci + 1) % 2][0])
        if cp_out[ci % 2] is not None:
            cp_out[ci % 2].wait()

        def vbody(vi, _):
            b16 = stage_in[0][pl.ds(vi * L, L)]
            sbin, perm = plsc.sort_key_val(b16, iota)
            prev = jnp.take(sbin, jnp.maximum(iota - 1, 0), mode="wrap")
            isstart = (sbin != prev) | (iota == 0)
            spos = plsc.cummax(jnp.where(isstart, iota, 0))
            occ_s = iota - spos
            nxt = jnp.take(sbin, jnp.minimum(iota + 1, L - 1), mode="wrap")
            islast = (sbin != nxt) | (iota == L - 1)
            sbc = jnp.where(sbin < 0, 135, sbin)
            base_s = plsc.load_gather(counters, [sbc])
            plsc.store_scatter(counters, [sbc], base_s + occ_s + 1, mask=islast)
            r_s = base_s + occ_s
            plsc.store_scatter(tmp16, [perm], r_s)
            r = tmp16[...]
            slot = (b16 * ROWS_PER_BAND + (r >> 5) * 1024 + t * 32 + (r & 31))
            ok = (b16 >= 0) & (r < CAP)
            dump = DUMP + ((t * 32 + (vi * L + iota)) & 1023)
            dest_v[pl.ds(vi * L, L)] = jnp.where(ok, slot, dump)
            rowpos = vi * L + iota
            lid = stage_in[1][pl.ds(vi * L, L)]
            plsc.store_scatter(rows_v, [rowpos, jnp.zeros((L,), jnp.int32)],
                               lid.astype(jnp.float32))
            for f in range(2, 10):
                val = stage_in[f][pl.ds(vi * L, L)]
                plsc.store_scatter(rows_v, [rowpos, jnp.full((L,), f - 1, jnp.int32)], val)
            return 0

        lax.fori_loop(0, K3_CHUNK // L, vbody, 0)
        cp_out[ci % 2] = pltpu.async_copy(rows_v, rows_hbm.at[dest_v], sem_out)

    for cp in cp_out:
        if cp is not None:
            cp.wait()
    pltpu.sync_copy(counters.at[pl.ds(0, 136)], counts_hbm.at[t])


@functools.partial(
    pl.kernel,
    out_type=(
        jax.ShapeDtypeStruct((NB, 5, BPX), jnp.float32),
        jax.ShapeDtypeStruct((NB, 5, 2048), jnp.float32),
    ),
    mesh=_sc_mesh,
    compiler_params=pltpu.CompilerParams(needs_layout_passes=False, use_tc_tiling_on_sc=False),
    scratch_types=[
        pltpu.VMEM((5 * PLANE,), jnp.float32),
        pltpu.VMEM((BPX,), jnp.float32),
        pltpu.VMEM((512, L), jnp.float32),
        pltpu.VMEM((512, L), jnp.float32),
        pltpu.VMEM((32, 136), jnp.int32),
        pltpu.SemaphoreType.DMA,
        pltpu.SemaphoreType.DMA,
    ],
)
def _raster_kernel(rows_hbm, counts_hbm, main_hbm, halo_hbm,
                   splat, zband, rows_va, rows_vb, counts_v, sem_a, sem_b):
    t = lax.axis_index("s") * 2 + lax.axis_index("c")
    iota = lax.iota(jnp.int32, L)
    pltpu.sync_copy(counts_hbm, counts_v)

    def band_body(k, _):
        b = t + 32 * k

        def do_band():
            bandbase = b * ROWS_PER_BAND

            def zzero(i, _):
                zband[pl.ds(i * L, L)] = jnp.full((L,), 100.0, jnp.float32)
                return 0

            lax.fori_loop(0, BPX // L, zzero, 0)

            def szero(i, _):
                splat[pl.ds(i * L, L)] = jnp.zeros((L,), jnp.float32)
                return 0

            lax.fori_loop(0, 5 * PLANE // L, szero, 0)

            c0 = plsc.load_gather(counts_v, [iota, jnp.full((L,), b, jnp.int32)])
            c1 = plsc.load_gather(counts_v, [iota + 16, jnp.full((L,), b, jnp.int32)])
            cmax = jnp.max(jnp.maximum(c0, c1))
            cmax = jnp.minimum(cmax, CAP)
            n512 = ((cmax + 31) // 32) * 2

            def pipelined(process):
                def proc(ref, c5):
                    def body(vi, _):
                        process(ref, c5, vi)
                        return 0
                    lax.fori_loop(0, 32, body, 0)

                def fetch(c5, ref, sem):
                    pltpu.async_copy(
                        rows_hbm.at[pl.ds(bandbase + c5 * 512, 512)], ref, sem)

                def waitbuf(ref, sem):
                    pltpu.make_async_copy(
                        rows_hbm.at[pl.ds(bandbase, 512)], ref, sem).wait()

                fetch(0, rows_va, sem_a)

                def pair(i, _):
                    c0 = 2 * i
                    waitbuf(rows_va, sem_a)
                    fetch(c0 + 1, rows_vb, sem_b)
                    proc(rows_va, c0)
                    waitbuf(rows_vb, sem_b)
                    fetch(c0 + 2, rows_va, sem_a)
                    pl.when(c0 + 1 < n512)(lambda: proc(rows_vb, c0 + 1))
                    return 0

                lax.fori_loop(0, (n512 + 1) // 2, pair, 0)
                waitbuf(rows_va, sem_a)

            def zvec(rows_v, c5, vi):
                if True:
                    jvec = c5 * 512 + vi * L + iota
                    t16 = (jvec >> 5) & 31
                    r16 = (jvec >> 10) * 32 + (jvec & 31)
                    cnt = plsc.load_gather(counts_v, [t16, jnp.full((L,), b, jnp.int32)])
                    ok = r16 < jnp.minimum(cnt, CAP)
                    loc = vi * L + iota
                    lid = plsc.load_gather(rows_v, [loc, jnp.zeros((L,), jnp.int32)])
                    z = plsc.load_gather(rows_v, [loc, jnp.ones((L,), jnp.int32)])
                    lidc = jnp.clip(lid.astype(jnp.int32), 0, BPX - 1)
                    zm = jnp.where(ok, z, 1e9)
                    sl, sz = plsc.sort_key_val(lidc, zm)
                    for kk in (1, 2, 4, 8):
                        pk = jnp.maximum(iota - kk, 0)
                        same = jnp.take(sl, pk, mode="wrap") == sl
                        cand = jnp.take(sz, pk, mode="wrap")
                        sz = jnp.where(same & (iota >= kk), jnp.minimum(sz, cand), sz)
                    nxt = jnp.take(sl, jnp.minimum(iota + 1, L - 1), mode="wrap")
                    islast = (sl != nxt) | (iota == L - 1)
                    cur = plsc.load_gather(zband, [sl])
                    plsc.store_scatter(zband, [sl], jnp.minimum(sz, cur), mask=islast)

            pipelined(zvec)

            def svec(rows_v, c5, vi):
                if True:
                    jvec = c5 * 512 + vi * L + iota
                    t16 = (jvec >> 5) & 31
                    r16 = (jvec >> 10) * 32 + (jvec & 31)
                    cnt = plsc.load_gather(counts_v, [t16, jnp.full((L,), b, jnp.int32)])
                    ok = r16 < jnp.minimum(cnt, CAP)
                    loc = vi * L + iota

                    def fld(f):
                        return plsc.load_gather(rows_v, [loc, jnp.full((L,), f, jnp.int32)])

                    lid = fld(0)
                    z = fld(1)
                    lidc = jnp.clip(lid.astype(jnp.int32), 0, BPX - 1)
                    minz = plsc.load_gather(zband, [lidc])
                    vis = ok & (z <= minz + 0.05)
                    zero = jnp.zeros((L,), jnp.float32)
                    wa = jnp.where(vis, fld(2), zero)
                    wb = jnp.where(vis, fld(3), zero)
                    wc = jnp.where(vis, fld(4), zero)
                    wd = jnp.where(vis, fld(5), zero)
                    r = jnp.where(vis, fld(6), zero)
                    g = jnp.where(vis, fld(7), zero)
                    bl = jnp.where(vis, fld(8), zero)
                    zf = jnp.where(vis, z, zero)
                    feats = (r, g, bl, zf)
                    for off, wgt in ((0, wa), (W, wb), (1, wc), (W + 1, wd)):
                        cidx = lidc + off
                        for ch in range(4):
                            plsc.addupdate_scatter(
                                splat, [cidx + ch * PLANE], wgt * feats[ch])
                        plsc.addupdate_scatter(splat, [cidx + 4 * PLANE], wgt)

            pipelined(svec)

            for ch in range(5):
                pltpu.sync_copy(splat.at[pl.ds(ch * PLANE, BPX)], main_hbm.at[b, ch])
                pltpu.sync_copy(splat.at[pl.ds(ch * PLANE + BPX, 2048)],
                                halo_hbm.at[b, ch])

        pl.when(b < NB)(do_band)
        return 0

    lax.fori_loop(0, 5, band_body, 0)


def _normalize_kernel(main, halo, out):
    pid = pl.program_id(0)
    m = main[...].reshape(5, BPX)
    hp = halo[...].reshape(5, 2048)
    factor = jnp.where(pid > 0, 1.0, 0.0)
    hpad = jnp.pad(hp * factor, ((0, 0), (0, BPX - 2048)))
    acc = m + hpad
    tw = acc[4] + 1e-6
    inv = 1.0 / tw
    rr = jnp.clip(acc[0] * inv, 0.0, 1.0)
    gg = jnp.clip(acc[1] * inv, 0.0, 1.0)
    bb = jnp.clip(acc[2] * inv, 0.0, 1.0)
    dd = acc[3] * inv
    out[...] = jnp.stack([rr, gg, bb, dd], axis=0).reshape(1, 4, BPX)


def _run_normalize(main, halo):
    return pl.pallas_call(
        _normalize_kernel,
        grid=(NB,),
        in_specs=[
            pl.BlockSpec((1, 5, BPX), lambda i: (i, 0, 0)),
            pl.BlockSpec((1, 5, 2048), lambda i: (jnp.maximum(i - 1, 0), 0, 0)),
        ],
        out_specs=pl.BlockSpec((1, 4, BPX), lambda i: (i, 0, 0)),
        out_shape=jax.ShapeDtypeStruct((NB, 4, BPX), jnp.float32),
    )(main, halo)


def kernel(means, colors, opacities, scales, quats, viewmat, K, height, width):
    n = means.shape[0]
    pad = NP - n
    mx = jnp.concatenate([means[:, 0], jnp.zeros((pad,), jnp.float32)])
    my = jnp.concatenate([means[:, 1], jnp.zeros((pad,), jnp.float32)])
    mz = jnp.concatenate([means[:, 2], jnp.full((pad,), -100.0, jnp.float32)])
    cr = jnp.concatenate([colors[:, 0], jnp.zeros((pad,), jnp.float32)])
    cg = jnp.concatenate([colors[:, 1], jnp.zeros((pad,), jnp.float32)])
    cb = jnp.concatenate([colors[:, 2], jnp.zeros((pad,), jnp.float32)])
    shp = (NP // 128, 128)
    R = viewmat[:3, :3]
    tv = viewmat[:3, 3]
    prm = jnp.concatenate([
        R.reshape(9), tv.reshape(3),
        jnp.stack([K[0, 0], K[1, 1], K[0, 2], K[1, 2]]),
    ]).astype(jnp.float32)
    outs = _run_project(mx.reshape(shp), my.reshape(shp), mz.reshape(shp),
                        cr.reshape(shp), cg.reshape(shp), cb.reshape(shp), prm)
    flat = [o.reshape(NP) for o in outs]
    rows, counts = _bin_kernel(*flat)
    main, halo = _raster_kernel(rows, counts)
    outp = _run_normalize(main, halo)
    return jnp.transpose(outp.reshape(NB, 4, 8, W), (0, 2, 3, 1)).reshape(H, W, 4)
